# Initial kernel scaffold; baseline (speedup 1.0000x reference)
import jax
import jax.numpy as jnp
from jax.experimental import pallas as pl


def kernel(atom_type, aa_type, aa_pos, atom_table, residue_table, pos_table, W1, b1, W2, b2):
    raise NotImplementedError("write your pallas kernel here")


# speedup vs baseline: 4.9822x; 4.9822x over previous
import functools

import jax
import jax.numpy as jnp
from jax.experimental import pallas as pl

N_ATOM, N_RES, N_POS = 55, 21, 24
C = 128
H = 3 * C
TOKEN_BLOCK = 1024


def _prep_body(epad_ref, w1_ref, out_ref):
    out_ref[:] = jnp.dot(epad_ref[:], w1_ref[:],
                         preferred_element_type=jnp.float32)


def _main_body(atom_ref, res_ref, pos_ref, comb_ref, b1_ref, w2_ref, b2_ref,
               out_ref):
    t = atom_ref.shape[0]
    iota = jax.lax.broadcasted_iota(jnp.int32, (t, C), 1)
    oh = ((iota == atom_ref[:]).astype(jnp.float32)
          + (iota == res_ref[:] + N_ATOM).astype(jnp.float32)
          + (iota == pos_ref[:] + (N_ATOM + N_RES)).astype(jnp.float32))
    h = jnp.dot(oh, comb_ref[:], preferred_element_type=jnp.float32)
    h = h + b1_ref[:]
    g = 0.5 * h * (1.0 + jax.lax.erf(h * 0.7071067811865476))
    out_ref[:] = jnp.dot(g, w2_ref[:],
                         preferred_element_type=jnp.float32) + b2_ref[:]


@functools.partial(jax.jit, static_argnames=())
def kernel(atom_type, aa_type, aa_pos, atom_table, residue_table, pos_table,
           W1, b1, W2, b2):
    b, l = atom_type.shape
    n = b * l

    epad = jnp.zeros((C, H), dtype=jnp.float32)
    epad = epad.at[0:N_ATOM, 0:C].set(atom_table)
    epad = epad.at[N_ATOM:N_ATOM + N_RES, C:2 * C].set(residue_table)
    epad = epad.at[N_ATOM + N_RES:N_ATOM + N_RES + N_POS, 2 * C:3 * C].set(
        pos_table)

    combined = pl.pallas_call(
        _prep_body,
        out_shape=jax.ShapeDtypeStruct((C, H), jnp.float32),
    )(epad, W1)

    t = TOKEN_BLOCK
    grid = (n // t,)
    idx_spec = pl.BlockSpec((t, 1), lambda i: (i, 0))
    full = lambda shape: pl.BlockSpec(shape, lambda i: (0, 0))

    out = pl.pallas_call(
        _main_body,
        grid=grid,
        in_specs=[
            idx_spec, idx_spec, idx_spec,
            full((C, H)),
            full((1, H)),
            full((H, C)),
            full((1, C)),
        ],
        out_specs=pl.BlockSpec((t, C), lambda i: (i, 0)),
        out_shape=jax.ShapeDtypeStruct((n, C), jnp.float32),
    )(atom_type.reshape(n, 1), aa_type.reshape(n, 1), aa_pos.reshape(n, 1),
      combined, b1.reshape(1, H), W2, b2.reshape(1, C))

    return out.reshape(b, l, C)


# speedup vs baseline: 5.0434x; 1.0123x over previous
import functools

import jax
import jax.numpy as jnp
from jax.experimental import pallas as pl

N_ATOM, N_RES, N_POS = 55, 21, 24
RES_OFF, POS_OFF = 64, 96
C = 128
H = 3 * C
TOKEN_BLOCK = 1024


def _prep_body(atom_ref, res_ref, pos_ref, w1_ref, w2_ref,
               comb_ref, w2h_ref):
    comb_ref[:] = jnp.zeros((C, H), dtype=jnp.float32)
    comb_ref[0:N_ATOM, :] = jnp.dot(
        atom_ref[:], w1_ref[0:C, :], preferred_element_type=jnp.float32)
    comb_ref[RES_OFF:RES_OFF + N_RES, :] = jnp.dot(
        res_ref[:], w1_ref[C:2 * C, :], preferred_element_type=jnp.float32)
    comb_ref[POS_OFF:POS_OFF + N_POS, :] = jnp.dot(
        pos_ref[:], w1_ref[2 * C:3 * C, :],
        preferred_element_type=jnp.float32)
    w2h_ref[:] = w2_ref[:] * 0.5


def _main_body(atom_ref, res_ref, pos_ref, comb_ref, b1_ref, w2h_ref, b2_ref,
               out_ref):
    t = atom_ref.shape[0]
    iota = jax.lax.broadcasted_iota(jnp.int32, (t, C), 1)
    hit = ((iota == atom_ref[:])
           | (iota == res_ref[:] + RES_OFF)
           | (iota == pos_ref[:] + POS_OFF))
    oh = hit.astype(jnp.float32)
    h = jnp.dot(oh, comb_ref[:], preferred_element_type=jnp.float32)
    h = h + b1_ref[:]
    g = h * (1.0 + jax.lax.erf(h * 0.7071067811865476))
    out_ref[:] = jnp.dot(g, w2h_ref[:],
                         preferred_element_type=jnp.float32) + b2_ref[:]


@functools.partial(jax.jit, static_argnames=())
def kernel(atom_type, aa_type, aa_pos, atom_table, residue_table, pos_table,
           W1, b1, W2, b2):
    b, l = atom_type.shape
    n = b * l

    combined, w2_half = pl.pallas_call(
        _prep_body,
        out_shape=(jax.ShapeDtypeStruct((C, H), jnp.float32),
                   jax.ShapeDtypeStruct((H, C), jnp.float32)),
    )(atom_table, residue_table, pos_table, W1, W2)

    t = TOKEN_BLOCK
    grid = (n // t,)
    idx_spec = pl.BlockSpec((t, 1), lambda i: (i, 0))
    full = lambda shape: pl.BlockSpec(shape, lambda i: (0, 0))

    out = pl.pallas_call(
        _main_body,
        grid=grid,
        in_specs=[
            idx_spec, idx_spec, idx_spec,
            full((C, H)),
            full((1, H)),
            full((H, C)),
            full((1, C)),
        ],
        out_specs=pl.BlockSpec((t, C), lambda i: (i, 0)),
        out_shape=jax.ShapeDtypeStruct((n, C), jnp.float32),
    )(atom_type.reshape(n, 1), aa_type.reshape(n, 1), aa_pos.reshape(n, 1),
      combined, b1.reshape(1, H), w2_half, b2.reshape(1, C))

    return out.reshape(b, l, C)


# speedup vs baseline: 11.0500x; 2.1910x over previous
import functools

import jax
import jax.numpy as jnp
from jax.experimental import pallas as pl

N_ATOM, N_RES, N_POS = 55, 21, 24
RES_OFF, POS_OFF = 64, 96
C = 128
H = 3 * C
ROW_BLOCK = 64


def _prep_body(atom_ref, res_ref, pos_ref, w1_ref, w2_ref,
               comb_ref, w2h_ref):
    comb_ref[:] = jnp.zeros((C, H), dtype=jnp.float32)
    comb_ref[0:N_ATOM, :] = jnp.dot(
        atom_ref[:], w1_ref[0:C, :], preferred_element_type=jnp.float32)
    comb_ref[RES_OFF:RES_OFF + N_RES, :] = jnp.dot(
        res_ref[:], w1_ref[C:2 * C, :], preferred_element_type=jnp.float32)
    comb_ref[POS_OFF:POS_OFF + N_POS, :] = jnp.dot(
        pos_ref[:], w1_ref[2 * C:3 * C, :],
        preferred_element_type=jnp.float32)
    w2h_ref[:] = w2_ref[:] * 0.5


def _main_body(code_ref, comb_ref, b1_ref, w2h_ref, b2_ref, out_ref):
    r, l, _ = out_ref.shape
    t = r * l
    code = code_ref[:]
    atom = jnp.bitwise_and(code, 127)
    res = jnp.bitwise_and(jnp.right_shift(code, 7), 31)
    pos = jnp.right_shift(code, 12)
    iota = jax.lax.broadcasted_iota(jnp.int32, (t, C), 1)
    hit = ((iota == atom)
           | (iota == res + RES_OFF)
           | (iota == pos + POS_OFF))
    oh = hit.astype(jnp.float32)
    h = jnp.dot(oh, comb_ref[:], preferred_element_type=jnp.float32)
    h = h + b1_ref[:]
    g = h * (1.0 + jax.lax.erf(h * 0.7071067811865476))
    g2 = jnp.dot(g, w2h_ref[:],
                 preferred_element_type=jnp.float32) + b2_ref[:]
    for k in range(r):
        out_ref[k] = g2[k * l:(k + 1) * l, :]


@functools.partial(jax.jit, static_argnames=())
def kernel(atom_type, aa_type, aa_pos, atom_table, residue_table, pos_table,
           W1, b1, W2, b2):
    b, l = atom_type.shape
    n = b * l

    combined, w2_half = pl.pallas_call(
        _prep_body,
        out_shape=(jax.ShapeDtypeStruct((C, H), jnp.float32),
                   jax.ShapeDtypeStruct((H, C), jnp.float32)),
    )(atom_table, residue_table, pos_table, W1, W2)

    packed = (atom_type + (aa_type << 7) + (aa_pos << 12)).reshape(n, 1)

    r = ROW_BLOCK
    t = r * l
    grid = (b // r,)
    full = lambda shape: pl.BlockSpec(shape, lambda i: (0, 0))

    out = pl.pallas_call(
        _main_body,
        grid=grid,
        in_specs=[
            pl.BlockSpec((t, 1), lambda i: (i, 0)),
            full((C, H)),
            full((1, H)),
            full((H, C)),
            full((1, C)),
        ],
        out_specs=pl.BlockSpec((r, l, C), lambda i: (i, 0, 0)),
        out_shape=jax.ShapeDtypeStruct((b, l, C), jnp.float32),
    )(packed, combined, b1.reshape(1, H), w2_half, b2.reshape(1, C))

    return out


# speedup vs baseline: 11.4482x; 1.0360x over previous
import functools

import jax
import jax.numpy as jnp
from jax.experimental import pallas as pl

N_ATOM, N_RES, N_POS = 55, 21, 24
RES_OFF, POS_OFF = 64, 96
B1_ROW = 120
C = 128
H = 3 * C
ROW_BLOCK = 128


def _prep_body(atom_ref, res_ref, pos_ref, w1_ref, b1_ref, w2_ref,
               comb_ref, w2h_ref):
    ca = jnp.dot(atom_ref[:], w1_ref[0:C, :],
                 preferred_element_type=jnp.float32)
    cr = jnp.dot(res_ref[:], w1_ref[C:2 * C, :],
                 preferred_element_type=jnp.float32)
    cp = jnp.dot(pos_ref[:], w1_ref[2 * C:3 * C, :],
                 preferred_element_type=jnp.float32)
    z = lambda k: jnp.zeros((k, H), dtype=jnp.float32)
    pieces = [ca, z(RES_OFF - N_ATOM), cr, z(POS_OFF - RES_OFF - N_RES), cp,
              z(B1_ROW - POS_OFF - N_POS), b1_ref[:], z(C - B1_ROW - 1)]
    comb = jnp.concatenate([p for p in pieces if p.shape[0] > 0], axis=0)
    comb_ref[:] = comb.astype(jnp.bfloat16)
    w2h_ref[:] = (w2_ref[:] * 0.5).astype(jnp.bfloat16)


def _bcast(sel, mask_j, ones_c, idx_ref):
    v = idx_ref[:].astype(jnp.float32).astype(jnp.bfloat16)
    y = jnp.dot(sel, v, preferred_element_type=jnp.float32)
    return jnp.dot(y.astype(jnp.bfloat16) * mask_j, ones_c,
                   preferred_element_type=jnp.float32)


def _main_body(atom_ref, res_ref, pos_ref, sel_ref, maskj_ref,
               comb_ref, w2h_ref, b2_ref, out_ref):
    r, l, _ = out_ref.shape
    t = r * l
    sel = sel_ref[:]
    mask_j = maskj_ref[:]
    ones_c = jnp.ones((l, C), dtype=jnp.bfloat16)
    ab = _bcast(sel, mask_j, ones_c, atom_ref)
    rb = _bcast(sel, mask_j, ones_c, res_ref)
    pb = _bcast(sel, mask_j, ones_c, pos_ref)
    iota = jax.lax.broadcasted_iota(
        jnp.int32, (t, C), 1).astype(jnp.float32)
    hit = ((iota == ab)
           | (iota == rb + 64.0)
           | (iota == pb + 96.0)
           | (iota == float(B1_ROW)))
    oh = hit.astype(jnp.bfloat16)
    h = jnp.dot(oh, comb_ref[:], preferred_element_type=jnp.float32)
    g = h * (1.0 + jax.lax.erf(h * 0.7071067811865476))
    g2 = jnp.dot(g.astype(jnp.bfloat16), w2h_ref[:],
                 preferred_element_type=jnp.float32) + b2_ref[:]
    for k in range(r):
        out_ref[k] = g2[k * l:(k + 1) * l, :]


@functools.partial(jax.jit, static_argnames=())
def kernel(atom_type, aa_type, aa_pos, atom_table, residue_table, pos_table,
           W1, b1, W2, b2):
    b, l = atom_type.shape

    combined, w2_half = pl.pallas_call(
        _prep_body,
        out_shape=(jax.ShapeDtypeStruct((C, H), jnp.bfloat16),
                   jax.ShapeDtypeStruct((H, C), jnp.bfloat16)),
    )(atom_table, residue_table, pos_table, W1, b1.reshape(1, H), W2)

    r = ROW_BLOCK
    t = r * l
    grid = (b // r,)

    tt = jnp.arange(t, dtype=jnp.int32)[:, None]
    rr = jnp.arange(r, dtype=jnp.int32)[None, :]
    sel = ((tt >= l * rr) & (tt < l * rr + l)).astype(jnp.bfloat16)
    mask_j = (jnp.arange(l, dtype=jnp.int32)[None, :]
              == tt % l).astype(jnp.bfloat16)

    idx_spec = pl.BlockSpec((r, l), lambda i: (i, 0))
    full = lambda shape: pl.BlockSpec(shape, lambda i: (0, 0))

    out = pl.pallas_call(
        _main_body,
        grid=grid,
        in_specs=[
            idx_spec, idx_spec, idx_spec,
            full((t, r)),
            full((t, l)),
            full((C, H)),
            full((H, C)),
            full((1, C)),
        ],
        out_specs=pl.BlockSpec((r, l, C), lambda i: (i, 0, 0)),
        out_shape=jax.ShapeDtypeStruct((b, l, C), jnp.float32),
    )(atom_type, aa_type, aa_pos, sel, mask_j,
      combined, w2_half, b2.reshape(1, C))

    return out
